# baseline (speedup 1.0000x reference)
import functools

import jax
import jax.numpy as jnp
from jax import lax
from jax.experimental import pallas as pl
from jax.experimental.pallas import tpu as pltpu
from jax.experimental.pallas import tpu_sc as plsc

_NW = 32
_CHUNK = 128


_DEPTH = 6
_AHEAD = 3


def _gather_body(nreal, cpw, idx_hbm, data_hbm, out_hbm, idx_v, *ring):
    wid = lax.axis_index("s") * 2 + lax.axis_index("c")
    base = wid * cpw
    pltpu.sync_copy(idx_hbm.at[wid], idx_v)

    rows = ring[:_DEPTH]
    gsem = ring[_DEPTH:2 * _DEPTH]
    wsem = ring[2 * _DEPTH:]

    def fire_gather(j, slot):
        pltpu.make_async_copy(
            data_hbm.at[idx_v.at[j]], rows[slot], gsem[slot]
        ).start()

    def wait_gather(j, slot):
        pltpu.make_async_copy(
            data_hbm.at[idx_v.at[j]], rows[slot], gsem[slot]
        ).wait()

    def fire_write(c, slot):
        pltpu.make_async_copy(
            rows[slot], out_hbm.at[pl.ds(c * _CHUNK, _CHUNK)], wsem[slot]
        ).start()

    def wait_write(slot):
        pltpu.make_async_copy(
            rows[slot], out_hbm.at[pl.ds(0, _CHUNK)], wsem[slot]
        ).wait()

    for jj in range(_AHEAD):
        @pl.when(base + jj < nreal)
        def _(jj=jj):
            fire_gather(jj, jj)

    def body(i, carry):
        for t in range(_DEPTH):
            j = _DEPTH * i + t
            c = base + j
            aslot = (t + _AHEAD) % _DEPTH
            @pl.when(jnp.logical_and(j + _AHEAD >= _DEPTH,
                                     c + _AHEAD - _DEPTH < nreal))
            def _():
                wait_write(aslot)

            @pl.when(jnp.logical_and(j + _AHEAD < cpw,
                                     c + _AHEAD < nreal))
            def _():
                fire_gather(j + _AHEAD, aslot)

            @pl.when(c < nreal)
            def _():
                wait_gather(j, t)
                fire_write(c, t)
        return carry

    lax.fori_loop(0, cpw // _DEPTH, body, 0)

    for jj in range(cpw - (_DEPTH - _AHEAD), cpw):
        @pl.when(base + jj < nreal)
        def _(jj=jj):
            wait_write(jj % _DEPTH)


def _sc_gather(idx, data, nreal, cpw):
    cin = data.shape[1]
    mesh = plsc.VectorSubcoreMesh(
        core_axis_name="c", subcore_axis_name="s", num_cores=2, num_subcores=16
    )
    body = functools.partial(_gather_body, nreal, cpw)
    return pl.kernel(
        body,
        out_type=jax.ShapeDtypeStruct((nreal * _CHUNK, cin), data.dtype),
        mesh=mesh,
        scratch_types=(
            [pltpu.VMEM((cpw, _CHUNK), jnp.int32)]
            + [pltpu.VMEM((_CHUNK, cin), data.dtype)] * _DEPTH
            + [pltpu.SemaphoreType.DMA] * (2 * _DEPTH)
        ),
    )(idx, data)


def _mm_body(nk, g_ref, w_ref, o_ref):
    acc = jnp.zeros(o_ref.shape, jnp.float32)
    for k in range(nk):
        acc += jnp.dot(
            g_ref[k].astype(jnp.bfloat16), w_ref[k],
            preferred_element_type=jnp.float32,
        )
    o_ref[...] = acc


def _tc_matmul(g3, wt, n_out, bm):
    nk, _, cin = g3.shape
    cout = wt.shape[2]
    return pl.pallas_call(
        functools.partial(_mm_body, nk),
        grid=(n_out // bm,),
        in_specs=[
            pl.BlockSpec((nk, bm, cin), lambda m: (0, m, 0)),
            pl.BlockSpec((nk, cin, cout), lambda m: (0, 0, 0)),
        ],
        out_specs=pl.BlockSpec((bm, cout), lambda m: (m, 0)),
        out_shape=jax.ShapeDtypeStruct((n_out, cout), jnp.float32),
        compiler_params=pltpu.CompilerParams(
            dimension_semantics=("arbitrary",),
        ),
    )(g3, wt)


_REP = 16


def _conv_piece(nbr_piece, data_rep, n_orig, wt, bm):
    npiece, kd = nbr_piece.shape
    cin = data_rep.shape[1]

    npad = ((npiece + _CHUNK - 1) // _CHUNK) * _CHUNK
    nchunk = kd * npad // _CHUNK
    cpw = (nchunk + _NW - 1) // _NW
    cpw = ((cpw + _DEPTH - 1) // _DEPTH) * _DEPTH

    idx = jnp.pad(nbr_piece, ((0, npad - npiece), (0, 0))).T
    idx = idx.reshape(nchunk, _CHUNK)
    idx = jnp.pad(idx, ((0, _NW * cpw - nchunk), (0, 0)))
    idx = idx.reshape(_NW, cpw, _CHUNK)

    off = (jnp.arange(_NW, dtype=jnp.int32) % _REP) * n_orig
    idx = idx + off.reshape(_NW, 1, 1)

    gathered = _sc_gather(idx, data_rep, nchunk, cpw)
    g3 = gathered.reshape(kd, npad, cin)
    return _tc_matmul(g3, wt, npiece, bm)


def kernel(data, neighbour, inv_neighbour, weights):
    n, cin = data.shape
    nbr = neighbour.astype(jnp.int32)
    wt = jnp.transpose(weights, (1, 2, 0)).astype(jnp.bfloat16)
    data_rep = jnp.broadcast_to(data, (_REP, n, cin)).reshape(_REP * n, cin)

    return _conv_piece(nbr, data_rep, n, wt, 400)


# speedup vs baseline: 1.0456x; 1.0456x over previous
import functools

import jax
import jax.numpy as jnp
from jax import lax
from jax.experimental import pallas as pl
from jax.experimental.pallas import tpu as pltpu
from jax.experimental.pallas import tpu_sc as plsc

_NW = 32
_CHUNK = 128


_DEPTH = 6
_AHEAD = 4


def _gather_body(nreal, cpw, idx_hbm, data_hbm, out_hbm, idx_v, *ring):
    wid = lax.axis_index("s") * 2 + lax.axis_index("c")
    base = wid * cpw
    pltpu.sync_copy(idx_hbm.at[wid], idx_v)

    rows = ring[:_DEPTH]
    gsem = ring[_DEPTH:2 * _DEPTH]
    wsem = ring[2 * _DEPTH:]

    def fire_gather(j, slot):
        pltpu.make_async_copy(
            data_hbm.at[idx_v.at[j]], rows[slot], gsem[slot]
        ).start()

    def wait_gather(j, slot):
        pltpu.make_async_copy(
            data_hbm.at[idx_v.at[j]], rows[slot], gsem[slot]
        ).wait()

    def fire_write(c, slot):
        pltpu.make_async_copy(
            rows[slot], out_hbm.at[pl.ds(c * _CHUNK, _CHUNK)], wsem[slot]
        ).start()

    def wait_write(slot):
        pltpu.make_async_copy(
            rows[slot], out_hbm.at[pl.ds(0, _CHUNK)], wsem[slot]
        ).wait()

    for jj in range(_AHEAD):
        @pl.when(base + jj < nreal)
        def _(jj=jj):
            fire_gather(jj, jj)

    def body(i, carry):
        for t in range(_DEPTH):
            j = _DEPTH * i + t
            c = base + j
            aslot = (t + _AHEAD) % _DEPTH
            @pl.when(jnp.logical_and(j + _AHEAD >= _DEPTH,
                                     c + _AHEAD - _DEPTH < nreal))
            def _():
                wait_write(aslot)

            @pl.when(jnp.logical_and(j + _AHEAD < cpw,
                                     c + _AHEAD < nreal))
            def _():
                fire_gather(j + _AHEAD, aslot)

            @pl.when(c < nreal)
            def _():
                wait_gather(j, t)
                fire_write(c, t)
        return carry

    lax.fori_loop(0, cpw // _DEPTH, body, 0)

    for jj in range(cpw - (_DEPTH - _AHEAD), cpw):
        @pl.when(base + jj < nreal)
        def _(jj=jj):
            wait_write(jj % _DEPTH)


def _sc_gather(idx, data, nreal, cpw):
    cin = data.shape[1]
    mesh = plsc.VectorSubcoreMesh(
        core_axis_name="c", subcore_axis_name="s", num_cores=2, num_subcores=16
    )
    body = functools.partial(_gather_body, nreal, cpw)
    return pl.kernel(
        body,
        out_type=jax.ShapeDtypeStruct((nreal * _CHUNK, cin), data.dtype),
        mesh=mesh,
        scratch_types=(
            [pltpu.VMEM((cpw, _CHUNK), jnp.int32)]
            + [pltpu.VMEM((_CHUNK, cin), data.dtype)] * _DEPTH
            + [pltpu.SemaphoreType.DMA] * (2 * _DEPTH)
        ),
    )(idx, data)


def _mm_body(nk, g_ref, w_ref, o_ref):
    acc = jnp.zeros(o_ref.shape, jnp.float32)
    for k in range(nk):
        acc += jnp.dot(
            g_ref[k].astype(jnp.bfloat16), w_ref[k],
            preferred_element_type=jnp.float32,
        )
    o_ref[...] = acc


def _tc_matmul(g3, wt, n_out, bm):
    nk, _, cin = g3.shape
    cout = wt.shape[2]
    return pl.pallas_call(
        functools.partial(_mm_body, nk),
        grid=(n_out // bm,),
        in_specs=[
            pl.BlockSpec((nk, bm, cin), lambda m: (0, m, 0)),
            pl.BlockSpec((nk, cin, cout), lambda m: (0, 0, 0)),
        ],
        out_specs=pl.BlockSpec((bm, cout), lambda m: (m, 0)),
        out_shape=jax.ShapeDtypeStruct((n_out, cout), jnp.float32),
        compiler_params=pltpu.CompilerParams(
            dimension_semantics=("arbitrary",),
        ),
    )(g3, wt)


_REP = 8


def _conv_piece(nbr_piece, data_rep, n_orig, wt, bm):
    npiece, kd = nbr_piece.shape
    cin = data_rep.shape[1]

    npad = ((npiece + _CHUNK - 1) // _CHUNK) * _CHUNK
    nchunk = kd * npad // _CHUNK
    cpw = (nchunk + _NW - 1) // _NW
    cpw = ((cpw + _DEPTH - 1) // _DEPTH) * _DEPTH

    idx = jnp.pad(nbr_piece, ((0, npad - npiece), (0, 0))).T
    idx = idx.reshape(nchunk, _CHUNK)
    idx = jnp.pad(idx, ((0, _NW * cpw - nchunk), (0, 0)))
    idx = idx.reshape(_NW, cpw, _CHUNK)

    off = (jnp.arange(_NW, dtype=jnp.int32) % _REP) * n_orig
    idx = idx + off.reshape(_NW, 1, 1)

    gathered = _sc_gather(idx, data_rep, nchunk, cpw)
    g3 = gathered.reshape(kd, npad, cin)
    return _tc_matmul(g3, wt, npiece, bm)


def kernel(data, neighbour, inv_neighbour, weights):
    n, cin = data.shape
    nbr = neighbour.astype(jnp.int32)
    wt = jnp.transpose(weights, (1, 2, 0)).astype(jnp.bfloat16)
    data_rep = jnp.broadcast_to(data, (_REP, n, cin)).reshape(_REP * n, cin)

    return _conv_piece(nbr, data_rep, n, wt, 400)
